# Initial kernel scaffold; baseline (speedup 1.0000x reference)
import jax
import jax.numpy as jnp
from jax.experimental import pallas as pl


def kernel(x, edge_index_0, edge_index_1, edge_index_2, W_dl, b_dl, W_fl, b_fl, W_lin_0, b_lin_0, W_att_0, b_att_0, W_lin_1, b_lin_1, W_att_1, b_att_1, W_lin_2, b_lin_2, W_att_2, b_att_2, W_out, b_out):
    raise NotImplementedError("write your pallas kernel here")


# speedup vs baseline: 1.0173x; 1.0173x over previous
import jax
import jax.numpy as jnp
from jax.experimental import pallas as pl

_N = 10000
_HEAD = 4
_HD = 64
_DH = _HEAD * _HD
_BLK = 1000


def _proj_kernel(x_ref, wcat_ref, bcat_ref, z_ref, out1_ref, out2_ref):
    o1 = jnp.dot(x_ref[...], wcat_ref[...], preferred_element_type=jnp.float32)
    o1 = o1 + bcat_ref[...]
    out1_ref[...] = o1
    out2_ref[...] = jnp.dot(o1, z_ref[...], preferred_element_type=jnp.float32)


def _out_kernel(h_ref, w_ref, b_ref, o_ref):
    o_ref[...] = (
        jnp.dot(h_ref[...], w_ref[...], preferred_element_type=jnp.float32)
        + b_ref[...]
    )


def _edge_stage(h, pv, qv, sgn, src, dst, b_att0):
    alpha = sgn[:, None] * pv[src] + qv[dst] + b_att0
    alpha = jnp.where(alpha >= 0, alpha, 0.01 * alpha)
    m = jax.ops.segment_max(alpha, dst, num_segments=_N)
    m = jnp.where(jnp.isfinite(m), m, 0.0)
    ex = jnp.exp(alpha - m[dst])
    denom = jax.ops.segment_sum(ex, dst, num_segments=_N)
    w = ex / (denom[dst] + 1e-16)
    coef = w * sgn[:, None]
    msg = coef[:, :, None] * h[src].reshape(-1, _HEAD, _HD)
    out = jax.ops.segment_sum(msg, dst, num_segments=_N)
    return out.reshape(_N, _DH)


def kernel(x, edge_index_0, edge_index_1, edge_index_2, W_dl, b_dl, W_fl, b_fl,
           W_lin_0, b_lin_0, W_att_0, b_att_0, W_lin_1, b_lin_1, W_att_1,
           b_att_1, W_lin_2, b_lin_2, W_att_2, b_att_2, W_out, b_out):
    Wcat = jnp.concatenate([W_lin_0, W_lin_1, W_lin_2], axis=1)
    bcat = jnp.concatenate([b_lin_0, b_lin_1, b_lin_2])[None, :]

    Z = jnp.zeros((3 * _DH, 128), dtype=jnp.float32)
    for r, W_att in enumerate((W_att_0, W_att_1, W_att_2)):
        Wa = W_att[:_HD, 0]
        Wb = W_att[_HD:, 0]
        for hd in range(_HEAD):
            rows = slice(_DH * r + _HD * hd, _DH * r + _HD * (hd + 1))
            Z = Z.at[rows, 8 * r + hd].set(Wa)
            Z = Z.at[rows, 8 * r + 4 + hd].set(Wb)

    grid = (_N // _BLK,)
    out1, out2 = pl.pallas_call(
        _proj_kernel,
        grid=grid,
        in_specs=[
            pl.BlockSpec((_BLK, 128), lambda i: (i, 0)),
            pl.BlockSpec((128, 3 * _DH), lambda i: (0, 0)),
            pl.BlockSpec((1, 3 * _DH), lambda i: (0, 0)),
            pl.BlockSpec((3 * _DH, 128), lambda i: (0, 0)),
        ],
        out_specs=[
            pl.BlockSpec((_BLK, 3 * _DH), lambda i: (i, 0)),
            pl.BlockSpec((_BLK, 128), lambda i: (i, 0)),
        ],
        out_shape=[
            jax.ShapeDtypeStruct((_N, 3 * _DH), jnp.float32),
            jax.ShapeDtypeStruct((_N, 128), jnp.float32),
        ],
    )(x, Wcat, bcat, Z)

    sp = x @ W_dl + b_dl

    hs = []
    for r, (ei, b_att) in enumerate(((edge_index_0, b_att_0),
                                     (edge_index_1, b_att_1),
                                     (edge_index_2, b_att_2))):
        src = ei[0]
        dst = ei[1]
        s_proj = sp[src]
        d_proj = sp[dst]
        e_feats = jnp.concatenate([s_proj, d_proj, s_proj - d_proj], axis=1)
        score = jnp.tanh((e_feats @ W_fl + b_fl)[:, 0])
        sgn = jnp.sign(score)

        h = out1[:, _DH * r:_DH * (r + 1)]
        pv = out2[:, 8 * r:8 * r + 4]
        qv = out2[:, 8 * r + 4:8 * r + 8]
        hs.append(_edge_stage(h, pv, qv, sgn, src, dst, b_att[0]))

    hcat = jnp.concatenate(hs, axis=1)
    out = pl.pallas_call(
        _out_kernel,
        grid=grid,
        in_specs=[
            pl.BlockSpec((_BLK, 3 * _DH), lambda i: (i, 0)),
            pl.BlockSpec((3 * _DH, _DH), lambda i: (0, 0)),
            pl.BlockSpec((1, _DH), lambda i: (0, 0)),
        ],
        out_specs=pl.BlockSpec((_BLK, _DH), lambda i: (i, 0)),
        out_shape=jax.ShapeDtypeStruct((_N, _DH), jnp.float32),
    )(hcat, W_out, b_out[None, :])
    return out
